# baseline (device time: 186412 ns/iter reference)
import jax
import jax.numpy as jnp
from jax import lax
from jax.experimental import pallas as pl
from jax.experimental.pallas import tpu as pltpu

N_Z = 4
K = 4
PIECE = 128
HALF = K * PIECE


def kernel(dy, W):
    m, k = dy.shape
    n, _ = W.shape

    def body(dy_ref, w_ref, out_ref, red_buf, red_send, red_recv, bc_send, bc_recv):
        my_x = lax.axis_index("x")
        my_y = lax.axis_index("y")
        my_z = lax.axis_index("z")

        def rows(f, p):
            return f * HALF + p * PIECE

        def compute(r0, nrows):
            out_ref[pl.ds(r0, nrows), :] = lax.dot_general(
                dy_ref[pl.ds(r0, nrows), :],
                w_ref[...],
                dimension_numbers=(((1,), (1,)), ((), ())),
                preferred_element_type=jnp.float32,
            )

        def rc(src, dst, ssem, rsem, tz):
            return pltpu.make_async_remote_copy(
                src_ref=src,
                dst_ref=dst,
                send_sem=ssem,
                recv_sem=rsem,
                device_id=(my_x, my_y, tz),
                device_id_type=pl.DeviceIdType.MESH,
            )

        barrier_sem = pltpu.get_barrier_semaphore()
        for nbr in ((my_z - 1) % N_Z, (my_z + 1) % N_Z):
            pl.semaphore_signal(
                barrier_sem,
                inc=1,
                device_id=(my_x, my_y, nbr),
                device_id_type=pl.DeviceIdType.MESH,
            )
        pl.semaphore_wait(barrier_sem, 2)

        def end_role(z):
            src_f = 0 if z == 3 else 1
            snk_f = 1 - src_f
            nbr = 2 if z == 3 else 1

            def _():
                pending = []
                compute(rows(src_f, 0), HALF)
                for p in range(K):
                    r = rc(
                        out_ref.at[pl.ds(rows(src_f, p), PIECE), :],
                        red_buf.at[src_f, p],
                        red_send.at[src_f, p],
                        red_recv.at[src_f, p],
                        nbr,
                    )
                    r.start()
                    pending.append(r)
                compute(rows(snk_f, 0), HALF)
                for p in range(K):
                    w = rc(
                        out_ref.at[pl.ds(rows(snk_f, p), PIECE), :],
                        red_buf.at[snk_f, p],
                        red_send.at[snk_f, p],
                        red_recv.at[snk_f, p],
                        nbr,
                    )
                    w.wait_recv()
                    out_ref[pl.ds(rows(snk_f, p), PIECE), :] += red_buf[snk_f, p]
                    b = rc(
                        out_ref.at[pl.ds(rows(snk_f, p), PIECE), :],
                        out_ref.at[pl.ds(rows(snk_f, p), PIECE), :],
                        bc_send.at[snk_f, p],
                        bc_recv.at[snk_f, p],
                        nbr,
                    )
                    b.start()
                    pending.append(b)
                for p in range(K):
                    w = rc(
                        out_ref.at[pl.ds(rows(src_f, p), PIECE), :],
                        out_ref.at[pl.ds(rows(src_f, p), PIECE), :],
                        bc_send.at[src_f, p],
                        bc_recv.at[src_f, p],
                        nbr,
                    )
                    w.wait_recv()
                for r in pending:
                    r.wait_send()

            return _

        def mid_role(z):
            up = {0: z + 1, 1: z - 1}
            down = {0: z - 1, 1: z + 1}

            def _():
                pending = []
                compute(rows(0, 0), 2 * PIECE)
                compute(rows(1, 0), 2 * PIECE)
                for p in range(K):
                    if p == 2:
                        compute(rows(0, 2), 2 * PIECE)
                        compute(rows(1, 2), 2 * PIECE)
                    for f in (0, 1):
                        w = rc(
                            out_ref.at[pl.ds(rows(f, p), PIECE), :],
                            red_buf.at[f, p],
                            red_send.at[f, p],
                            red_recv.at[f, p],
                            up[f],
                        )
                        w.wait_recv()
                        out_ref[pl.ds(rows(f, p), PIECE), :] += red_buf[f, p]
                        r = rc(
                            out_ref.at[pl.ds(rows(f, p), PIECE), :],
                            red_buf.at[f, p],
                            red_send.at[f, p],
                            red_recv.at[f, p],
                            down[f],
                        )
                        r.start()
                        pending.append(r)
                for p in range(K):
                    for f in (0, 1):
                        w = rc(
                            out_ref.at[pl.ds(rows(f, p), PIECE), :],
                            out_ref.at[pl.ds(rows(f, p), PIECE), :],
                            bc_send.at[f, p],
                            bc_recv.at[f, p],
                            down[f],
                        )
                        w.wait_recv()
                        b = rc(
                            out_ref.at[pl.ds(rows(f, p), PIECE), :],
                            out_ref.at[pl.ds(rows(f, p), PIECE), :],
                            bc_send.at[f, p],
                            bc_recv.at[f, p],
                            up[f],
                        )
                        b.start()
                        pending.append(b)
                for r in pending:
                    r.wait_send()

            return _

        pl.when(my_z == 0)(end_role(0))
        pl.when(my_z == 1)(mid_role(1))
        pl.when(my_z == 2)(mid_role(2))
        pl.when(my_z == 3)(end_role(3))

    return pl.pallas_call(
        body,
        out_shape=jax.ShapeDtypeStruct((m, n), jnp.float32),
        in_specs=[
            pl.BlockSpec(memory_space=pltpu.VMEM),
            pl.BlockSpec(memory_space=pltpu.VMEM),
        ],
        out_specs=pl.BlockSpec(memory_space=pltpu.VMEM),
        scratch_shapes=[
            pltpu.VMEM((2, K, PIECE, n), jnp.float32),
            pltpu.SemaphoreType.DMA((2, K)),
            pltpu.SemaphoreType.DMA((2, K)),
            pltpu.SemaphoreType.DMA((2, K)),
            pltpu.SemaphoreType.DMA((2, K)),
        ],
        compiler_params=pltpu.CompilerParams(collective_id=0),
    )(dy, W)


# device time: 68486 ns/iter; 2.7219x vs baseline; 2.7219x over previous
import jax
import jax.numpy as jnp
from jax import lax
from jax.experimental import pallas as pl
from jax.experimental.pallas import tpu as pltpu

N_Z = 4
QROWS = 256
SUB = 64


def kernel(dy, W):
    m, kdim = dy.shape
    n, _ = W.shape

    def body(
        dy_ref, w_ref, out_ref, zrs_buf,
        zrs_send, zrs_recv, zag_send, zag_recv,
        xy1_send, xy1_recv, xy2_send, xy2_recv,
    ):
        x = lax.axis_index("x")
        y = lax.axis_index("y")
        z = lax.axis_index("z")
        qrow = (2 * x + y) * QROWS
        sub_row = qrow + z * SUB

        def rc(src, dst, ssem, rsem, dev):
            return pltpu.make_async_remote_copy(
                src_ref=src, dst_ref=dst, send_sem=ssem, recv_sem=rsem,
                device_id=dev, device_id_type=pl.DeviceIdType.MESH,
            )

        out_ref[pl.ds(qrow, QROWS), :] = lax.dot_general(
            dy_ref[pl.ds(qrow, QROWS), :],
            w_ref[...],
            dimension_numbers=(((1,), (1,)), ((), ())),
            preferred_element_type=jnp.float32,
        )

        barrier_sem = pltpu.get_barrier_semaphore()
        for dz in (1, 2, 3):
            pl.semaphore_signal(
                barrier_sem, inc=1,
                device_id=(x, y, (z + dz) % N_Z),
                device_id_type=pl.DeviceIdType.MESH,
            )
        for dev in ((1 - x, y, z), (x, 1 - y, z)):
            pl.semaphore_signal(
                barrier_sem, inc=1, device_id=dev,
                device_id_type=pl.DeviceIdType.MESH,
            )
        pl.semaphore_wait(barrier_sem, 5)

        pending = []

        for dz in (1, 2, 3):
            tz = (z + dz) % N_Z
            r = rc(
                out_ref.at[pl.ds(qrow + tz * SUB, SUB), :],
                zrs_buf.at[3 - dz],
                zrs_send.at[dz - 1],
                zrs_recv.at[3 - dz],
                (x, y, tz),
            )
            r.start()
            pending.append(r)
        for j in range(3):
            rc(
                out_ref.at[pl.ds(sub_row, SUB), :],
                zrs_buf.at[j],
                zrs_send.at[j],
                zrs_recv.at[j],
                (x, y, z),
            ).wait_recv()
        out_ref[pl.ds(sub_row, SUB), :] += (
            zrs_buf[0] + zrs_buf[1] + zrs_buf[2]
        )

        for dz in (1, 2, 3):
            tz = (z + dz) % N_Z
            r = rc(
                out_ref.at[pl.ds(sub_row, SUB), :],
                out_ref.at[pl.ds(sub_row, SUB), :],
                zag_send.at[dz - 1],
                zag_recv.at[3 - dz],
                (x, y, tz),
            )
            r.start()
            pending.append(r)
        for j in range(3):
            rc(
                out_ref.at[pl.ds(sub_row, SUB), :],
                out_ref.at[pl.ds(sub_row, SUB), :],
                zag_send.at[j],
                zag_recv.at[j],
                (x, y, z),
            ).wait_recv()

        for slot, dev in ((0, (1 - x, y, z)), (1, (x, 1 - y, z))):
            r = rc(
                out_ref.at[pl.ds(qrow, QROWS), :],
                out_ref.at[pl.ds(qrow, QROWS), :],
                xy1_send.at[slot],
                xy1_recv.at[slot],
                dev,
            )
            r.start()
            pending.append(r)

        qy_row = (2 * x + (1 - y)) * QROWS
        rc(
            out_ref.at[pl.ds(qy_row, QROWS), :],
            out_ref.at[pl.ds(qy_row, QROWS), :],
            xy1_send.at[1],
            xy1_recv.at[1],
            (x, 1 - y, z),
        ).wait_recv()
        r = rc(
            out_ref.at[pl.ds(qy_row, QROWS), :],
            out_ref.at[pl.ds(qy_row, QROWS), :],
            xy2_send.at[0],
            xy2_recv.at[0],
            (1 - x, y, z),
        )
        r.start()
        pending.append(r)

        qx_row = (2 * (1 - x) + y) * QROWS
        rc(
            out_ref.at[pl.ds(qx_row, QROWS), :],
            out_ref.at[pl.ds(qx_row, QROWS), :],
            xy1_send.at[0],
            xy1_recv.at[0],
            (1 - x, y, z),
        ).wait_recv()
        qd_row = (2 * (1 - x) + (1 - y)) * QROWS
        rc(
            out_ref.at[pl.ds(qd_row, QROWS), :],
            out_ref.at[pl.ds(qd_row, QROWS), :],
            xy2_send.at[0],
            xy2_recv.at[0],
            (1 - x, y, z),
        ).wait_recv()

        for r in pending:
            r.wait_send()

    return pl.pallas_call(
        body,
        out_shape=jax.ShapeDtypeStruct((m, n), jnp.float32),
        in_specs=[
            pl.BlockSpec(memory_space=pltpu.VMEM),
            pl.BlockSpec(memory_space=pltpu.VMEM),
        ],
        out_specs=pl.BlockSpec(memory_space=pltpu.VMEM),
        scratch_shapes=[
            pltpu.VMEM((3, SUB, n), jnp.float32),
            pltpu.SemaphoreType.DMA((3,)),
            pltpu.SemaphoreType.DMA((3,)),
            pltpu.SemaphoreType.DMA((3,)),
            pltpu.SemaphoreType.DMA((3,)),
            pltpu.SemaphoreType.DMA((2,)),
            pltpu.SemaphoreType.DMA((2,)),
            pltpu.SemaphoreType.DMA((1,)),
            pltpu.SemaphoreType.DMA((1,)),
        ],
        compiler_params=pltpu.CompilerParams(collective_id=0),
    )(dy, W)


# device time: 62007 ns/iter; 3.0063x vs baseline; 1.1045x over previous
import jax
import jax.numpy as jnp
from jax import lax
from jax.experimental import pallas as pl
from jax.experimental.pallas import tpu as pltpu

N_Z = 4
QROWS = 256
SUB = 64


def kernel(dy, W):
    m, kdim = dy.shape
    n, _ = W.shape

    def body(
        dy_ref, w_ref, out_ref, zrs_buf,
        zrs_send, zrs_recv, zag_send, zag_recv,
        xy1_send, xy1_recv, xy2_send, xy2_recv,
    ):
        x = lax.axis_index("x")
        y = lax.axis_index("y")
        z = lax.axis_index("z")
        qrow = (2 * x + y) * QROWS
        qx_row = (2 * (1 - x) + y) * QROWS
        qy_row = (2 * x + (1 - y)) * QROWS
        qd_row = (2 * (1 - x) + (1 - y)) * QROWS

        def sub_of(base, j):
            return base + ((z + j) % N_Z) * SUB

        def rc(src, dst, ssem, rsem, dev):
            return pltpu.make_async_remote_copy(
                src_ref=src, dst_ref=dst, send_sem=ssem, recv_sem=rsem,
                device_id=dev, device_id_type=pl.DeviceIdType.MESH,
            )

        barrier_sem = pltpu.get_barrier_semaphore()
        for dz in (1, 2, 3):
            pl.semaphore_signal(
                barrier_sem, inc=1,
                device_id=(x, y, (z + dz) % N_Z),
                device_id_type=pl.DeviceIdType.MESH,
            )
        for dev in ((1 - x, y, z), (x, 1 - y, z)):
            pl.semaphore_signal(
                barrier_sem, inc=1, device_id=dev,
                device_id_type=pl.DeviceIdType.MESH,
            )

        out_ref[pl.ds(qrow, QROWS), :] = lax.dot_general(
            dy_ref[pl.ds(qrow, QROWS), :],
            w_ref[...],
            dimension_numbers=(((1,), (1,)), ((), ())),
            preferred_element_type=jnp.float32,
        )

        pl.semaphore_wait(barrier_sem, 5)

        pending = []

        for dz in (1, 2, 3):
            tz = (z + dz) % N_Z
            r = rc(
                out_ref.at[pl.ds(sub_of(qrow, dz), SUB), :],
                zrs_buf.at[3 - dz],
                zrs_send.at[dz - 1],
                zrs_recv.at[3 - dz],
                (x, y, tz),
            )
            r.start()
            pending.append(r)
        my_sub = sub_of(qrow, 0)
        for j in range(3):
            rc(
                out_ref.at[pl.ds(my_sub, SUB), :],
                zrs_buf.at[j],
                zrs_send.at[j],
                zrs_recv.at[j],
                (x, y, z),
            ).wait_recv()
            out_ref[pl.ds(my_sub, SUB), :] += zrs_buf[j]

        for dz in (1, 2, 3):
            tz = (z + dz) % N_Z
            r = rc(
                out_ref.at[pl.ds(my_sub, SUB), :],
                out_ref.at[pl.ds(my_sub, SUB), :],
                zag_send.at[dz - 1],
                zag_recv.at[3 - dz],
                (x, y, tz),
            )
            r.start()
            pending.append(r)

        def xy1(j):
            r0 = sub_of(qrow, j)
            for link, dev in ((0, (1 - x, y, z)), (1, (x, 1 - y, z))):
                r = rc(
                    out_ref.at[pl.ds(r0, SUB), :],
                    out_ref.at[pl.ds(r0, SUB), :],
                    xy1_send.at[link, j],
                    xy1_recv.at[link, j],
                    dev,
                )
                r.start()
                pending.append(r)

        xy1(0)
        for j in (1, 3, 2):
            rc(
                out_ref.at[pl.ds(my_sub, SUB), :],
                out_ref.at[pl.ds(my_sub, SUB), :],
                zag_send.at[j - 1],
                zag_recv.at[j - 1],
                (x, y, z),
            ).wait_recv()
            xy1(j)

        for j in (0, 1, 3, 2):
            r0 = sub_of(qy_row, j)
            rc(
                out_ref.at[pl.ds(r0, SUB), :],
                out_ref.at[pl.ds(r0, SUB), :],
                xy1_send.at[1, j],
                xy1_recv.at[1, j],
                (x, 1 - y, z),
            ).wait_recv()
            r = rc(
                out_ref.at[pl.ds(r0, SUB), :],
                out_ref.at[pl.ds(r0, SUB), :],
                xy2_send.at[j],
                xy2_recv.at[j],
                (1 - x, y, z),
            )
            r.start()
            pending.append(r)

        for j in range(N_Z):
            rc(
                out_ref.at[pl.ds(sub_of(qx_row, j), SUB), :],
                out_ref.at[pl.ds(sub_of(qx_row, j), SUB), :],
                xy1_send.at[0, j],
                xy1_recv.at[0, j],
                (1 - x, y, z),
            ).wait_recv()
        for j in range(N_Z):
            rc(
                out_ref.at[pl.ds(sub_of(qd_row, j), SUB), :],
                out_ref.at[pl.ds(sub_of(qd_row, j), SUB), :],
                xy2_send.at[j],
                xy2_recv.at[j],
                (1 - x, y, z),
            ).wait_recv()

        for r in pending:
            r.wait_send()

    return pl.pallas_call(
        body,
        out_shape=jax.ShapeDtypeStruct((m, n), jnp.float32),
        in_specs=[
            pl.BlockSpec(memory_space=pltpu.VMEM),
            pl.BlockSpec(memory_space=pltpu.VMEM),
        ],
        out_specs=pl.BlockSpec(memory_space=pltpu.VMEM),
        scratch_shapes=[
            pltpu.VMEM((3, SUB, n), jnp.float32),
            pltpu.SemaphoreType.DMA((3,)),
            pltpu.SemaphoreType.DMA((3,)),
            pltpu.SemaphoreType.DMA((3,)),
            pltpu.SemaphoreType.DMA((3,)),
            pltpu.SemaphoreType.DMA((2, N_Z)),
            pltpu.SemaphoreType.DMA((2, N_Z)),
            pltpu.SemaphoreType.DMA((N_Z,)),
            pltpu.SemaphoreType.DMA((N_Z,)),
        ],
        compiler_params=pltpu.CompilerParams(collective_id=0),
    )(dy, W)


# device time: 60179 ns/iter; 3.0976x vs baseline; 1.0304x over previous
import jax
import jax.numpy as jnp
from jax import lax
from jax.experimental import pallas as pl
from jax.experimental.pallas import tpu as pltpu

N_Z = 4
QROWS = 256
SUB = 64


def kernel(dy, W):
    m, kdim = dy.shape
    n, _ = W.shape

    def body(
        dy_ref, w_ref, out_ref, zrs_buf,
        zrs_send, zrs_recv, zag_send, zag_recv,
        xy1_send, xy1_recv, xy2_send, xy2_recv,
    ):
        x = lax.axis_index("x")
        y = lax.axis_index("y")
        z = lax.axis_index("z")
        qrow = (2 * x + y) * QROWS
        qx_row = (2 * (1 - x) + y) * QROWS
        qy_row = (2 * x + (1 - y)) * QROWS
        qd_row = (2 * (1 - x) + (1 - y)) * QROWS

        def sub_of(base, j):
            return base + ((z + j) % N_Z) * SUB

        def rc(src, dst, ssem, rsem, dev):
            return pltpu.make_async_remote_copy(
                src_ref=src, dst_ref=dst, send_sem=ssem, recv_sem=rsem,
                device_id=dev, device_id_type=pl.DeviceIdType.MESH,
            )

        barrier_sem = pltpu.get_barrier_semaphore()
        for dz in (1, 2, 3):
            pl.semaphore_signal(
                barrier_sem, inc=1,
                device_id=(x, y, (z + dz) % N_Z),
                device_id_type=pl.DeviceIdType.MESH,
            )
        for dev in ((1 - x, y, z), (x, 1 - y, z), (1 - x, 1 - y, z)):
            pl.semaphore_signal(
                barrier_sem, inc=1, device_id=dev,
                device_id_type=pl.DeviceIdType.MESH,
            )

        out_ref[pl.ds(qrow, QROWS), :] = lax.dot_general(
            dy_ref[pl.ds(qrow, QROWS), :],
            w_ref[...],
            dimension_numbers=(((1,), (1,)), ((), ())),
            preferred_element_type=jnp.float32,
        )

        pl.semaphore_wait(barrier_sem, 6)

        pending = []

        for dz in (1, 2, 3):
            tz = (z + dz) % N_Z
            r = rc(
                out_ref.at[pl.ds(sub_of(qrow, dz), SUB), :],
                zrs_buf.at[3 - dz],
                zrs_send.at[dz - 1],
                zrs_recv.at[3 - dz],
                (x, y, tz),
            )
            r.start()
            pending.append(r)
        my_sub = sub_of(qrow, 0)
        for j in range(3):
            rc(
                out_ref.at[pl.ds(my_sub, SUB), :],
                zrs_buf.at[j],
                zrs_send.at[j],
                zrs_recv.at[j],
                (x, y, z),
            ).wait_recv()
            out_ref[pl.ds(my_sub, SUB), :] += zrs_buf[j]

        for dz in (1, 2, 3):
            tz = (z + dz) % N_Z
            r = rc(
                out_ref.at[pl.ds(my_sub, SUB), :],
                out_ref.at[pl.ds(my_sub, SUB), :],
                zag_send.at[dz - 1],
                zag_recv.at[3 - dz],
                (x, y, tz),
            )
            r.start()
            pending.append(r)

        def xy1(j):
            r0 = sub_of(qrow, j)
            for link, dev in ((0, (1 - x, y, z)), (1, (x, 1 - y, z))):
                r = rc(
                    out_ref.at[pl.ds(r0, SUB), :],
                    out_ref.at[pl.ds(r0, SUB), :],
                    xy1_send.at[link, j],
                    xy1_recv.at[link, j],
                    dev,
                )
                r.start()
                pending.append(r)
            r = rc(
                out_ref.at[pl.ds(r0, SUB), :],
                out_ref.at[pl.ds(r0, SUB), :],
                xy2_send.at[j],
                xy2_recv.at[j],
                (1 - x, 1 - y, z),
            )
            r.start()
            pending.append(r)

        xy1(0)
        for j in (1, 3, 2):
            rc(
                out_ref.at[pl.ds(my_sub, SUB), :],
                out_ref.at[pl.ds(my_sub, SUB), :],
                zag_send.at[j - 1],
                zag_recv.at[j - 1],
                (x, y, z),
            ).wait_recv()
            xy1(j)

        for j in range(N_Z):
            rc(
                out_ref.at[pl.ds(sub_of(qx_row, j), SUB), :],
                out_ref.at[pl.ds(sub_of(qx_row, j), SUB), :],
                xy1_send.at[0, j],
                xy1_recv.at[0, j],
                (1 - x, y, z),
            ).wait_recv()
            rc(
                out_ref.at[pl.ds(sub_of(qy_row, j), SUB), :],
                out_ref.at[pl.ds(sub_of(qy_row, j), SUB), :],
                xy1_send.at[1, j],
                xy1_recv.at[1, j],
                (x, 1 - y, z),
            ).wait_recv()
            rc(
                out_ref.at[pl.ds(sub_of(qd_row, j), SUB), :],
                out_ref.at[pl.ds(sub_of(qd_row, j), SUB), :],
                xy2_send.at[j],
                xy2_recv.at[j],
                (1 - x, 1 - y, z),
            ).wait_recv()

        for r in pending:
            r.wait_send()

    return pl.pallas_call(
        body,
        out_shape=jax.ShapeDtypeStruct((m, n), jnp.float32),
        in_specs=[
            pl.BlockSpec(memory_space=pltpu.VMEM),
            pl.BlockSpec(memory_space=pltpu.VMEM),
        ],
        out_specs=pl.BlockSpec(memory_space=pltpu.VMEM),
        scratch_shapes=[
            pltpu.VMEM((3, SUB, n), jnp.float32),
            pltpu.SemaphoreType.DMA((3,)),
            pltpu.SemaphoreType.DMA((3,)),
            pltpu.SemaphoreType.DMA((3,)),
            pltpu.SemaphoreType.DMA((3,)),
            pltpu.SemaphoreType.DMA((2, N_Z)),
            pltpu.SemaphoreType.DMA((2, N_Z)),
            pltpu.SemaphoreType.DMA((N_Z,)),
            pltpu.SemaphoreType.DMA((N_Z,)),
        ],
        compiler_params=pltpu.CompilerParams(collective_id=0),
    )(dy, W)
